# baseline (speedup 1.0000x reference)
import functools

import jax
import jax.numpy as jnp
from jax.experimental import pallas as pl
from jax.experimental.pallas import tpu as pltpu

N_LEFT = 8
N_RIGHT = 8
N_HEADS = 12
_BT = 128


def _attn_body(e_ref, isn_ref, wq_ref, wk_ref, wv_ref, wo_ref, o_ref, *, T, D):
    H = N_HEADS
    dh = D // H
    W = N_LEFT + N_RIGHT
    i = pl.program_id(1)
    t0 = i * _BT

    eh = e_ref[0, pl.ds(t0, _BT + W), :]
    ec = eh[N_LEFT:N_LEFT + _BT, :]

    ehb = eh.astype(jnp.bfloat16)
    q = jnp.dot(ehb[N_LEFT:N_LEFT + _BT, :], wq_ref[...],
                preferred_element_type=jnp.float32)
    k = jnp.dot(ehb, wk_ref[...], preferred_element_type=jnp.float32)
    v = jnp.dot(ehb, wv_ref[...], preferred_element_type=jnp.float32)

    qi = jax.lax.broadcasted_iota(jnp.int32, (_BT, _BT + W), 0)
    kj = jax.lax.broadcasted_iota(jnp.int32, (_BT, _BT + W), 1)
    pos_k = t0 + kj - N_LEFT
    mask = (kj >= qi) & (kj <= qi + W) & (pos_k >= 0) & (pos_k < T)

    scale = 1.0 / (dh ** 0.5)
    qb = q.astype(jnp.bfloat16)
    kb = k.astype(jnp.bfloat16)
    vb = v.astype(jnp.bfloat16)
    outs = []
    for h in range(H):
        sl = slice(h * dh, (h + 1) * dh)
        s = jax.lax.dot_general(qb[:, sl], kb[:, sl],
                                (((1,), (1,)), ((), ())),
                                preferred_element_type=jnp.float32)
        s = jnp.where(mask, s * scale, jnp.float32(-1e9))
        m = jnp.max(s, axis=1, keepdims=True)
        p = jnp.exp(s - m)
        p = p / jnp.sum(p, axis=1, keepdims=True)
        outs.append(jnp.dot(p.astype(jnp.bfloat16), vb[:, sl],
                            preferred_element_type=jnp.float32))
    attn = jnp.concatenate(outs, axis=1)
    a = jnp.dot(attn.astype(jnp.bfloat16), wo_ref[...],
                preferred_element_type=jnp.float32)

    msk = isn_ref[0] != 0
    o_ref[0] = jnp.where(msk, a, ec)


def kernel(embeds, is_numbers, Wq, Wk, Wv, Wo):
    B, T, D = embeds.shape
    W = N_LEFT + N_RIGHT
    e_pad = jnp.pad(embeds, ((0, 0), (N_LEFT, N_RIGHT), (0, 0)))
    isn = is_numbers.astype(jnp.int32).reshape(B, T, 1)
    grid = (B, T // _BT)
    wq, wk, wv, wo = (w.astype(jnp.bfloat16) for w in (Wq, Wk, Wv, Wo))
    return pl.pallas_call(
        functools.partial(_attn_body, T=T, D=D),
        grid=grid,
        in_specs=[
            pl.BlockSpec((1, T + W, D), lambda b, i: (b, 0, 0)),
            pl.BlockSpec((1, _BT, 1), lambda b, i: (b, i, 0)),
            pl.BlockSpec((D, D), lambda b, i: (0, 0)),
            pl.BlockSpec((D, D), lambda b, i: (0, 0)),
            pl.BlockSpec((D, D), lambda b, i: (0, 0)),
            pl.BlockSpec((D, D), lambda b, i: (0, 0)),
        ],
        out_specs=pl.BlockSpec((1, _BT, D), lambda b, i: (b, i, 0)),
        out_shape=jax.ShapeDtypeStruct((B, T, D), jnp.float32),
        compiler_params=pltpu.CompilerParams(
            dimension_semantics=("parallel", "arbitrary"),
        ),
    )(e_pad, isn, wq, wk, wv, wo)


# speedup vs baseline: 1.0773x; 1.0773x over previous
import functools

import jax
import jax.numpy as jnp
from jax.experimental import pallas as pl
from jax.experimental.pallas import tpu as pltpu

N_LEFT = 8
N_RIGHT = 8
N_HEADS = 12
_BT = 112
_NEG = -1e9
_CSHIFT = -30.0


def _attn_body(e_ref, isn_ref, w3_ref, wo_ref, o_ref, *, T, D):
    H = N_HEADS
    dh = D // H
    W = N_LEFT + N_RIGHT
    KW = _BT + W
    i = pl.program_id(1)
    t0 = i * _BT

    eh = e_ref[0, pl.ds(t0, KW), :].astype(jnp.bfloat16)
    qkv = jnp.dot(eh, w3_ref[...],
                  preferred_element_type=jnp.float32).astype(jnp.bfloat16)
    q = qkv[N_LEFT:N_LEFT + _BT, 0:D]
    k = qkv[:, D:2 * D]
    v = qkv[:, 2 * D:3 * D]

    qi = jax.lax.broadcasted_iota(jnp.int32, (_BT, KW), 0)
    kj = jax.lax.broadcasted_iota(jnp.int32, (_BT, KW), 1)
    pos_k = t0 + kj - N_LEFT
    mask = (kj >= qi) & (kj <= qi + W) & (pos_k >= 0) & (pos_k < T)
    bias = jnp.where(mask, jnp.float32(_CSHIFT), jnp.float32(_NEG))

    outs = []
    for h in range(H):
        sl = slice(h * dh, (h + 1) * dh)
        s = jax.lax.dot_general(q[:, sl], k[:, sl],
                                (((1,), (1,)), ((), ())),
                                preferred_element_type=jnp.float32)
        p = jnp.exp(s + bias)
        r = 1.0 / jnp.sum(p, axis=1, keepdims=True)
        o = jnp.dot(p.astype(jnp.bfloat16), v[:, sl],
                    preferred_element_type=jnp.float32)
        outs.append(o * r)
    attn = jnp.concatenate(outs, axis=1).astype(jnp.bfloat16)
    a = jnp.dot(attn, wo_ref[...], preferred_element_type=jnp.float32)

    ec = e_ref[0, pl.ds(t0 + N_LEFT, _BT), :]
    msk = isn_ref[0] != 0
    o_ref[0] = jnp.where(msk, a, ec)


def kernel(embeds, is_numbers, Wq, Wk, Wv, Wo):
    B, T, D = embeds.shape
    H = N_HEADS
    dh = D // H
    W = N_LEFT + N_RIGHT
    nblk = -(-T // _BT)
    Tp = nblk * _BT
    e_pad = jnp.pad(embeds, ((0, 0), (N_LEFT, Tp + W - N_LEFT - T), (0, 0)))
    isn = jnp.pad(is_numbers.astype(jnp.int32), ((0, 0), (0, Tp - T)))
    isn = isn.reshape(B, Tp, 1)
    scale = 1.0 / (dh ** 0.5)
    w3 = jnp.concatenate([Wq * scale, Wk, Wv], axis=1).astype(jnp.bfloat16)
    wo = Wo.astype(jnp.bfloat16)
    out = pl.pallas_call(
        functools.partial(_attn_body, T=T, D=D),
        grid=(B, nblk),
        in_specs=[
            pl.BlockSpec((1, Tp + W, D), lambda b, i: (b, 0, 0)),
            pl.BlockSpec((1, _BT, 1), lambda b, i: (b, i, 0)),
            pl.BlockSpec((D, 3 * D), lambda b, i: (0, 0)),
            pl.BlockSpec((D, D), lambda b, i: (0, 0)),
        ],
        out_specs=pl.BlockSpec((1, _BT, D), lambda b, i: (b, i, 0)),
        out_shape=jax.ShapeDtypeStruct((B, Tp, D), jnp.float32),
        compiler_params=pltpu.CompilerParams(
            dimension_semantics=("parallel", "arbitrary"),
        ),
    )(e_pad, isn, w3, wo)
    return out[:, :T, :]


# speedup vs baseline: 1.4398x; 1.3364x over previous
import functools

import jax
import jax.numpy as jnp
from jax.experimental import pallas as pl
from jax.experimental.pallas import tpu as pltpu

N_LEFT = 8
N_RIGHT = 8
N_HEADS = 12
_BT = 128
_KW = _BT + 16
_NEG = -1e9
_CSHIFT = -30.0


def _attn_body(e_ref, isn_ref, w3_ref, wo_ref, o_ref, *, T, D):
    H = N_HEADS
    dh = D // H
    i = pl.program_id(1)
    t0 = i * _BT
    h0 = jnp.maximum(0, jnp.minimum(t0 - N_LEFT, T - _KW))
    h0 = pl.multiple_of(h0, 8)
    delta = t0 - h0

    ec = e_ref[0, pl.ds(t0, _BT), :]
    eh = e_ref[0, pl.ds(h0, _KW), :].astype(jnp.bfloat16)
    q = jnp.dot(ec.astype(jnp.bfloat16), w3_ref[:, 0:D],
                preferred_element_type=jnp.float32).astype(jnp.bfloat16)
    kv = jnp.dot(eh, w3_ref[:, D:3 * D],
                 preferred_element_type=jnp.float32).astype(jnp.bfloat16)
    k = kv[:, 0:D]
    v = kv[:, D:2 * D]

    qi = jax.lax.broadcasted_iota(jnp.int32, (_BT, _KW), 0)
    kj = jax.lax.broadcasted_iota(jnp.int32, (_BT, _KW), 1)
    rel = kj - qi - delta
    mask = (rel >= -N_LEFT) & (rel <= N_RIGHT)
    bias = jnp.where(mask, jnp.float32(_CSHIFT), jnp.float32(_NEG))

    outs = []
    for h in range(H):
        sl = slice(h * dh, (h + 1) * dh)
        s = jax.lax.dot_general(q[:, sl], k[:, sl],
                                (((1,), (1,)), ((), ())),
                                preferred_element_type=jnp.float32)
        p = jnp.exp(s + bias)
        r = 1.0 / jnp.sum(p, axis=1, keepdims=True)
        o = jnp.dot(p.astype(jnp.bfloat16), v[:, sl],
                    preferred_element_type=jnp.float32)
        outs.append(o * r)
    attn = jnp.concatenate(outs, axis=1).astype(jnp.bfloat16)
    a = jnp.dot(attn, wo_ref[...], preferred_element_type=jnp.float32)

    msk = isn_ref[0] != 0
    o_ref[0] = jnp.where(msk, a, ec)


def kernel(embeds, is_numbers, Wq, Wk, Wv, Wo):
    B, T, D = embeds.shape
    dh = D // N_HEADS
    isn = is_numbers.astype(jnp.int32).reshape(B, T, 1)
    scale = 1.0 / (dh ** 0.5)
    w3 = jnp.concatenate([Wq * scale, Wk, Wv], axis=1).astype(jnp.bfloat16)
    wo = Wo.astype(jnp.bfloat16)
    return pl.pallas_call(
        functools.partial(_attn_body, T=T, D=D),
        grid=(B, T // _BT),
        in_specs=[
            pl.BlockSpec((1, T, D), lambda b, i: (b, 0, 0)),
            pl.BlockSpec((1, _BT, 1), lambda b, i: (b, i, 0)),
            pl.BlockSpec((D, 3 * D), lambda b, i: (0, 0)),
            pl.BlockSpec((D, D), lambda b, i: (0, 0)),
        ],
        out_specs=pl.BlockSpec((1, _BT, D), lambda b, i: (b, i, 0)),
        out_shape=jax.ShapeDtypeStruct((B, T, D), jnp.float32),
        compiler_params=pltpu.CompilerParams(
            dimension_semantics=("parallel", "arbitrary"),
        ),
    )(embeds, isn, w3, wo)
